# baseline (speedup 1.0000x reference)
import functools

import jax
import jax.numpy as jnp
from jax import lax
from jax.experimental import pallas as pl
from jax.experimental.pallas import tpu as pltpu
from jax.experimental.pallas import tpu_sc as plsc

NC = 2
NS = 16
NW = NC * NS
CH = 128
NBUF = 3
G = 2


@functools.lru_cache(maxsize=None)
def _make_lookup(B, V, D):
    assert B % (NW * CH) == 0
    bpw = B // NW
    k = bpw // CH
    assert k % G == 0
    k2 = k // G
    rounds = k2 // NBUF
    tail = k2 - rounds * NBUF

    mesh = plsc.VectorSubcoreMesh(core_axis_name="c", subcore_axis_name="s")

    @functools.partial(
        pl.kernel,
        mesh=mesh,
        out_type=jax.ShapeDtypeStruct((B, D), jnp.float32),
        scratch_types=[
            pltpu.VMEM((k, CH), jnp.int32),
            pltpu.VMEM((NBUF, G * CH, D), jnp.float32),
            pltpu.SemaphoreType.DMA((NBUF,)),
            pltpu.SemaphoreType.DMA((NBUF,)),
        ],
    )
    def lookup(idx_hbm, table_hbm, out_hbm, idx_v, rows_v, gsem, osem):
        wid = lax.axis_index("s") * NC + lax.axis_index("c")
        base = wid * bpw

        pltpu.sync_copy(idx_hbm.at[wid], idx_v)

        def g_copy(s, b, h):
            return pltpu.make_async_copy(
                table_hbm.at[idx_v.at[G * s + h]],
                rows_v.at[b, pl.ds(h * CH, CH)],
                gsem.at[b])

        def o_copy(s, b):
            return pltpu.make_async_copy(
                rows_v.at[b],
                out_hbm.at[pl.ds(base + s * G * CH, G * CH)],
                osem.at[b])

        for b in range(min(NBUF, k2)):
            for h in range(G):
                g_copy(b, b, h).start()

        def round_body(i, carry):
            s0 = i * NBUF
            for b in range(NBUF):
                s = s0 + b
                for h in range(G):
                    g_copy(s, b, h).wait()
                o_copy(s, b).start()
                @pl.when(s + NBUF < k2)
                def _():
                    o_copy(s, b).wait()
                    for h in range(G):
                        g_copy(s + NBUF, b, h).start()
            return carry

        lax.fori_loop(0, rounds, round_body, 0)

        for t in range(tail):
            s = rounds * NBUF + t
            for h in range(G):
                g_copy(s, t, h).wait()
            o_copy(s, t).start()

        for d in range(NBUF):
            s = k2 - NBUF + d
            o_copy(s, s % NBUF).wait()

    return lookup


def kernel(text, table):
    V, D = table.shape
    tokens = text.astype(jnp.int32).reshape(-1)
    B = tokens.shape[0]
    idx = tokens.reshape(NW, B // (NW * CH), CH)
    out = _make_lookup(B, V, D)(idx, table)
    return out.reshape(text.shape + (D,))


# speedup vs baseline: 1.0150x; 1.0150x over previous
import functools

import jax
import jax.numpy as jnp
from jax import lax
from jax.experimental import pallas as pl
from jax.experimental.pallas import tpu as pltpu
from jax.experimental.pallas import tpu_sc as plsc

NC = 2
NS = 16
NW = NC * NS
CH = 128
NBUF = 7


@functools.lru_cache(maxsize=None)
def _make_lookup(B, V, D):
    assert B % (NW * CH) == 0
    bpw = B // NW
    k = bpw // CH
    rounds = k // NBUF
    tail = k - rounds * NBUF

    mesh = plsc.VectorSubcoreMesh(core_axis_name="c", subcore_axis_name="s")

    @functools.partial(
        pl.kernel,
        mesh=mesh,
        out_type=jax.ShapeDtypeStruct((B, D), jnp.float32),
        scratch_types=[
            pltpu.VMEM((k, CH), jnp.int32),
            pltpu.VMEM((NBUF, CH, D), jnp.float32),
            pltpu.SemaphoreType.DMA((NBUF,)),
            pltpu.SemaphoreType.DMA((NBUF,)),
            pltpu.SemaphoreType.DMA,
        ],
    )
    def lookup(idx_hbm, table_hbm, out_hbm, idx_v, rows_v, gsem, osem, isem):
        wid = lax.axis_index("s") * NC + lax.axis_index("c")
        base = wid * bpw

        head = 8
        pltpu.sync_copy(idx_hbm.at[wid, pl.ds(0, head)],
                        idx_v.at[pl.ds(0, head)])
        rest = pltpu.make_async_copy(idx_hbm.at[wid, pl.ds(head, k - head)],
                                     idx_v.at[pl.ds(head, k - head)], isem)
        rest.start()

        def g_copy(j, b):
            return pltpu.make_async_copy(
                table_hbm.at[idx_v.at[j]], rows_v.at[b], gsem.at[b])

        def o_copy(j, b):
            return pltpu.make_async_copy(
                rows_v.at[b], out_hbm.at[pl.ds(base + j * CH, CH)], osem.at[b])

        for b in range(min(NBUF, k)):
            g_copy(b, b).start()

        rest.wait()

        def round_body(i, carry):
            j0 = i * NBUF
            for b in range(NBUF):
                j = j0 + b
                g_copy(j, b).wait()
                o_copy(j, b).start()
                @pl.when(j + NBUF < k)
                def _():
                    o_copy(j, b).wait()
                    g_copy(j + NBUF, b).start()
            return carry

        lax.fori_loop(0, rounds, round_body, 0)

        for t in range(tail):
            j = rounds * NBUF + t
            g_copy(j, t).wait()
            o_copy(j, t).start()

        for d in range(NBUF):
            j = k - NBUF + d
            o_copy(j, j % NBUF).wait()

    return lookup


def kernel(text, table):
    V, D = table.shape
    tokens = text.astype(jnp.int32).reshape(-1)
    B = tokens.shape[0]
    idx = tokens.reshape(NW, B // (NW * CH), CH)
    out = _make_lookup(B, V, D)(idx, table)
    return out.reshape(text.shape + (D,))
